# Initial kernel scaffold; baseline (speedup 1.0000x reference)
import jax
import jax.numpy as jnp
from jax.experimental import pallas as pl


def kernel(x, edge_index, edge_attr, W, W_edge, att_src, att_dst, att_edge, bias):
    raise NotImplementedError("write your pallas kernel here")


# speedup vs baseline: 19.8822x; 19.8822x over previous
import functools

import jax
import jax.numpy as jnp
from jax import lax
from jax.experimental import pallas as pl
from jax.experimental.pallas import tpu as pltpu
from jax.experimental.pallas import tpu_sc as plsc

H = 8
C = 32
HC = H * C
F_IN = 256
E_DIM = 16
H16 = 16
NC = 2
NS = 16
K = 128
HALF = HC // NC
STRIPE = 640
NP = NS * STRIPE


def _head_sum_matrix(n_in, n_heads, group):
    rows = lax.broadcasted_iota(jnp.int32, (n_in, n_heads), 0)
    cols = lax.broadcasted_iota(jnp.int32, (n_in, n_heads), 1)
    return jnp.where(rows // group == cols, 1.0, 0.0).astype(jnp.float32)


def _proj_body(x_ref, w_ref, asrc_ref, adst_ref, xw_ref, s_ref, d_ref):
    xw = lax.dot_general(x_ref[...], w_ref[...], (((1,), (1,)), ((), ())),
                         preferred_element_type=jnp.float32)
    xw_ref[...] = xw
    S = _head_sum_matrix(HC, H16, C)
    s_ref[...] = lax.dot_general(xw * asrc_ref[...], S,
                                 (((1,), (0,)), ((), ())),
                                 preferred_element_type=jnp.float32)
    d_ref[...] = lax.dot_general(xw * adst_ref[...], S,
                                 (((1,), (0,)), ((), ())),
                                 preferred_element_type=jnp.float32)


def _run_proj(x, W, asrc, adst, n, bn):
    grid = n // bn
    return pl.pallas_call(
        _proj_body,
        grid=(grid,),
        in_specs=[
            pl.BlockSpec((bn, F_IN), lambda i: (i, 0)),
            pl.BlockSpec((HC, F_IN), lambda i: (0, 0)),
            pl.BlockSpec((1, HC), lambda i: (0, 0)),
            pl.BlockSpec((1, HC), lambda i: (0, 0)),
        ],
        out_specs=[
            pl.BlockSpec((bn, HC), lambda i: (i, 0)),
            pl.BlockSpec((bn, H16), lambda i: (i, 0)),
            pl.BlockSpec((bn, H16), lambda i: (i, 0)),
        ],
        out_shape=[
            jax.ShapeDtypeStruct((n, HC), jnp.float32),
            jax.ShapeDtypeStruct((n, H16), jnp.float32),
            jax.ShapeDtypeStruct((n, H16), jnp.float32),
        ],
    )(x, W, asrc, adst)


def _edge_logit_body(ea_ref, we_ref, aedge_ref, e_ref):
    S = _head_sum_matrix(HC, H16, C)
    aw = we_ref[...] * aedge_ref[...]
    V = lax.dot_general(S, aw, (((0,), (0,)), ((), ())),
                        preferred_element_type=jnp.float32)
    e_ref[...] = lax.dot_general(ea_ref[...], V, (((1,), (1,)), ((), ())),
                                 preferred_element_type=jnp.float32)


def _run_edge_logits(ea_p, W_edge, aedge, epad, be):
    grid = epad // be
    return pl.pallas_call(
        _edge_logit_body,
        grid=(grid,),
        in_specs=[
            pl.BlockSpec((be, E_DIM), lambda i: (i, 0)),
            pl.BlockSpec((HC, E_DIM), lambda i: (0, 0)),
            pl.BlockSpec((HC, 1), lambda i: (0, 0)),
        ],
        out_specs=pl.BlockSpec((be, H16), lambda i: (i, 0)),
        out_shape=jax.ShapeDtypeStruct((epad, H16), jnp.float32),
    )(ea_p, W_edge, aedge)


def _sc_pass1_body(src_ref, dst_ref, e_ref, s_ref, d_ref, ones_ref, zeros_ref,
                   p_ref, den_ref, es_ref, cnt_ref,
                   den_sh, es_sh, cnt_sh,
                   sidx, didx, srow, drow, erow, prow, onesv, gsem,
                   nb):
    cid = lax.axis_index("c")
    sid = lax.axis_index("s")
    start = pl.multiple_of(sid * STRIPE, 8)

    pltpu.sync_copy(zeros_ref.at[pl.ds(start, STRIPE)],
                    den_sh.at[pl.ds(start, STRIPE)])
    pltpu.sync_copy(zeros_ref.at[pl.ds(start, STRIPE)],
                    es_sh.at[pl.ds(start, STRIPE)])
    pltpu.sync_copy(zeros_ref.at[pl.ds(start, STRIPE)],
                    cnt_sh.at[pl.ds(start, STRIPE)])
    pltpu.sync_copy(ones_ref, onesv)
    plsc.subcore_barrier()

    base = (cid * NS + sid) * (nb * K)

    def batch_body(b, carry):
        off = pl.multiple_of(base + b * K, 8)
        pltpu.sync_copy(src_ref.at[pl.ds(off, K)], sidx)
        pltpu.sync_copy(dst_ref.at[pl.ds(off, K)], didx)
        pltpu.sync_copy(e_ref.at[pl.ds(off, K)], erow)
        pltpu.async_copy(s_ref.at[sidx], srow, gsem).wait()
        pltpu.async_copy(d_ref.at[didx], drow, gsem).wait()

        def edge_p(j, c2):
            t = srow[j, :] + drow[j, :] + erow[j, :]
            t = jnp.where(t > 0.0, t, 0.2 * t)
            prow[j, :] = jnp.exp(t)
            return c2

        lax.fori_loop(0, K, edge_p, 0)
        pltpu.sync_copy(prow, p_ref.at[pl.ds(off, K)])
        pltpu.sync_copy(prow, den_sh.at[didx], add=True)
        pltpu.sync_copy(erow, es_sh.at[didx], add=True)
        pltpu.sync_copy(onesv, cnt_sh.at[didx], add=True)
        return carry

    lax.fori_loop(0, nb, batch_body, 0)
    plsc.subcore_barrier()

    pltpu.sync_copy(den_sh.at[pl.ds(start, STRIPE)],
                    den_ref.at[cid, pl.ds(start, STRIPE)])
    pltpu.sync_copy(es_sh.at[pl.ds(start, STRIPE)],
                    es_ref.at[cid, pl.ds(start, STRIPE)])
    pltpu.sync_copy(cnt_sh.at[pl.ds(start, STRIPE)],
                    cnt_ref.at[cid, pl.ds(start, STRIPE)])


def _run_sc_pass1(src_p, dst_p, e, s, d_pad, epad):
    nb = epad // (NC * NS * K)
    ones16 = jnp.ones((K, H16), jnp.float32)
    zeros16 = jnp.zeros((NP, H16), jnp.float32)
    mesh = plsc.VectorSubcoreMesh(core_axis_name="c", subcore_axis_name="s")
    body = functools.partial(_sc_pass1_body, nb=nb)
    fn = pl.kernel(
        body,
        out_type=[
            jax.ShapeDtypeStruct((epad, H16), jnp.float32),
            jax.ShapeDtypeStruct((NC, NP, H16), jnp.float32),
            jax.ShapeDtypeStruct((NC, NP, H16), jnp.float32),
            jax.ShapeDtypeStruct((NC, NP, H16), jnp.float32),
        ],
        mesh=mesh,
        compiler_params=pltpu.CompilerParams(use_tc_tiling_on_sc=False),
        scratch_types=[
            pltpu.VMEM_SHARED((NP, H16), jnp.float32),
            pltpu.VMEM_SHARED((NP, H16), jnp.float32),
            pltpu.VMEM_SHARED((NP, H16), jnp.float32),
            pltpu.VMEM((K,), jnp.int32),
            pltpu.VMEM((K,), jnp.int32),
            pltpu.VMEM((K, H16), jnp.float32),
            pltpu.VMEM((K, H16), jnp.float32),
            pltpu.VMEM((K, H16), jnp.float32),
            pltpu.VMEM((K, H16), jnp.float32),
            pltpu.VMEM((K, H16), jnp.float32),
            pltpu.SemaphoreType.DMA,
        ],
    )
    return fn(src_p, dst_p, e, s, d_pad, ones16, zeros16)


def _sc_pass2_body(src_ref, dst_ref, p_ref, xw2_ref, zeros_ref,
                   acc_ref,
                   acc_sh, sidx, didx, gidx, prow, xrow, gsem,
                   nb):
    cid = lax.axis_index("c")
    sid = lax.axis_index("s")
    start = pl.multiple_of(sid * STRIPE, 8)

    pltpu.sync_copy(zeros_ref.at[pl.ds(start, STRIPE)],
                    acc_sh.at[pl.ds(start, STRIPE)])
    plsc.subcore_barrier()

    base = sid * (nb * K)

    def batch_body(b, carry):
        off = pl.multiple_of(base + b * K, 8)
        pltpu.sync_copy(src_ref.at[pl.ds(off, K)], sidx)
        pltpu.sync_copy(dst_ref.at[pl.ds(off, K)], didx)
        pltpu.sync_copy(p_ref.at[pl.ds(off, K)], prow)

        def idx_body(i, c2):
            g = sidx[pl.ds(i * 16, 16)] * 2 + cid
            gidx[pl.ds(i * 16, 16)] = g
            return c2

        lax.fori_loop(0, K // 16, idx_body, 0)
        pltpu.async_copy(xw2_ref.at[gidx], xrow, gsem).wait()

        def edge_body(j, c2):
            pv = prow[j, :]
            for t in range(HALF // 16):
                w = jnp.where(cid == 0, pv[t // 2], pv[H // NC + t // 2])
                xs = xrow[j, pl.ds(t * 16, 16)]
                xrow[j, pl.ds(t * 16, 16)] = xs * w
            return c2

        lax.fori_loop(0, K, edge_body, 0)
        pltpu.sync_copy(xrow, acc_sh.at[didx], add=True)
        return carry

    lax.fori_loop(0, nb, batch_body, 0)
    plsc.subcore_barrier()
    pltpu.sync_copy(acc_sh.at[pl.ds(start, STRIPE)],
                    acc_ref.at[cid, pl.ds(start, STRIPE)])


def _run_sc_pass2(src_p, dst_p, p, xw2, epad):
    nb = epad // (NS * K)
    zerosw = jnp.zeros((NP, HALF), jnp.float32)
    mesh = plsc.VectorSubcoreMesh(core_axis_name="c", subcore_axis_name="s")
    body = functools.partial(_sc_pass2_body, nb=nb)
    fn = pl.kernel(
        body,
        out_type=jax.ShapeDtypeStruct((NC, NP, HALF), jnp.float32),
        mesh=mesh,
        compiler_params=pltpu.CompilerParams(use_tc_tiling_on_sc=False),
        scratch_types=[
            pltpu.VMEM_SHARED((NP, HALF), jnp.float32),
            pltpu.VMEM((K,), jnp.int32),
            pltpu.VMEM((K,), jnp.int32),
            pltpu.VMEM((K,), jnp.int32),
            pltpu.VMEM((K, H16), jnp.float32),
            pltpu.VMEM((K, HALF), jnp.float32),
            pltpu.SemaphoreType.DMA,
        ],
    )
    return fn(src_p, dst_p, p, xw2, zerosw)


def _combine_body(acc0_ref, acc1_ref, den0_ref, den1_ref, es0_ref, es1_ref,
                  cnt0_ref, cnt1_ref, s_ref, d_ref, xw_ref, bias_ref, out_ref):
    den = den0_ref[...] + den1_ref[...]
    es = es0_ref[...] + es1_ref[...]
    deg = cnt0_ref[...] + cnt1_ref[...]
    eloop = es / jnp.maximum(deg, 1.0)
    t = s_ref[...] + d_ref[...] + eloop
    t = jnp.where(t > 0.0, t, 0.2 * t)
    aself = jnp.exp(t)
    scale = 1.0 / ((den + aself + 1e-16) * (deg + 1.0))
    Ex = _head_sum_matrix(HC, H16, C)
    aself_w = lax.dot_general(aself, Ex, (((1,), (1,)), ((), ())),
                              preferred_element_type=jnp.float32)
    scale_w = lax.dot_general(scale, Ex, (((1,), (1,)), ((), ())),
                              preferred_element_type=jnp.float32)
    acc = jnp.concatenate([acc0_ref[...], acc1_ref[...]], axis=1)
    o = (acc + aself_w * xw_ref[...]) * scale_w + bias_ref[...]
    out_ref[...] = jnp.where(o > 0.0, o, jnp.exp(o) - 1.0)


def _run_combine(acc0, acc1, den2, es2, cnt2, s, d, xw, bias_row, n, bn):
    grid = n // bn
    nspec = lambda w: pl.BlockSpec((bn, w), lambda i: (i, 0))
    return pl.pallas_call(
        _combine_body,
        grid=(grid,),
        in_specs=[
            nspec(HALF), nspec(HALF),
            nspec(H16), nspec(H16), nspec(H16), nspec(H16),
            nspec(H16), nspec(H16),
            nspec(H16), nspec(H16),
            nspec(HC),
            pl.BlockSpec((1, HC), lambda i: (0, 0)),
        ],
        out_specs=pl.BlockSpec((bn, HC), lambda i: (i, 0)),
        out_shape=jax.ShapeDtypeStruct((n, HC), jnp.float32),
    )(acc0, acc1, den2[0], den2[1], es2[0], es2[1], cnt2[0], cnt2[1],
      s, d, xw, bias_row)


def kernel(x, edge_index, edge_attr, W, W_edge, att_src, att_dst, att_edge,
           bias):
    n = x.shape[0]
    e_cnt = edge_index.shape[1]
    chunk = NC * NS * K
    epad = ((e_cnt + chunk - 1) // chunk) * chunk
    pad = epad - e_cnt

    src_p = jnp.concatenate(
        [edge_index[0].astype(jnp.int32), jnp.zeros((pad,), jnp.int32)])
    dst_p = jnp.concatenate(
        [edge_index[1].astype(jnp.int32), jnp.full((pad,), n, jnp.int32)])
    ea_p = jnp.concatenate(
        [edge_attr, jnp.zeros((pad, E_DIM), jnp.float32)], axis=0)

    asrc = att_src.reshape(1, HC)
    adst = att_dst.reshape(1, HC)
    aedge = att_edge.reshape(HC, 1)
    bias_row = bias.reshape(1, HC)

    xw, s, d = _run_proj(x, W, asrc, adst, n, bn=2000)
    e = _run_edge_logits(ea_p, W_edge, aedge, epad, be=4096)

    d_pad = jnp.concatenate([d, jnp.zeros((1, H16), jnp.float32)], axis=0)
    p, den2, es2, cnt2 = _run_sc_pass1(src_p, dst_p, e, s, d_pad, epad)

    xw2 = xw.reshape(n * NC, HALF)
    acc2 = _run_sc_pass2(src_p, dst_p, p, xw2, epad)

    return _run_combine(acc2[0, :n], acc2[1, :n], den2[:, :n], es2[:, :n],
                        cnt2[:, :n], s, d, xw, bias_row, n, bn=2000)


# speedup vs baseline: 24.3965x; 1.2270x over previous
import functools

import jax
import jax.numpy as jnp
from jax import lax
from jax.experimental import pallas as pl
from jax.experimental.pallas import tpu as pltpu
from jax.experimental.pallas import tpu_sc as plsc

H = 8
C = 32
HC = H * C
F_IN = 256
E_DIM = 16
H16 = 16
NC = 2
NS = 16
K = 128
HALF = HC // NC
STRIPE = 640
NP = NS * STRIPE


def _head_sum_matrix(n_in, n_heads, group):
    rows = lax.broadcasted_iota(jnp.int32, (n_in, n_heads), 0)
    cols = lax.broadcasted_iota(jnp.int32, (n_in, n_heads), 1)
    return jnp.where(rows // group == cols, 1.0, 0.0).astype(jnp.float32)


def _proj_body(x_ref, w_ref, asrc_ref, adst_ref, xw_ref, s_ref, d_ref):
    xw = lax.dot_general(x_ref[...], w_ref[...], (((1,), (1,)), ((), ())),
                         preferred_element_type=jnp.float32)
    xw_ref[...] = xw
    S = _head_sum_matrix(HC, H16, C)
    s_ref[...] = lax.dot_general(xw * asrc_ref[...], S,
                                 (((1,), (0,)), ((), ())),
                                 preferred_element_type=jnp.float32)
    d_ref[...] = lax.dot_general(xw * adst_ref[...], S,
                                 (((1,), (0,)), ((), ())),
                                 preferred_element_type=jnp.float32)


def _run_proj(x, W, asrc, adst, n, bn):
    grid = n // bn
    return pl.pallas_call(
        _proj_body,
        grid=(grid,),
        in_specs=[
            pl.BlockSpec((bn, F_IN), lambda i: (i, 0)),
            pl.BlockSpec((HC, F_IN), lambda i: (0, 0)),
            pl.BlockSpec((1, HC), lambda i: (0, 0)),
            pl.BlockSpec((1, HC), lambda i: (0, 0)),
        ],
        out_specs=[
            pl.BlockSpec((bn, HC), lambda i: (i, 0)),
            pl.BlockSpec((bn, H16), lambda i: (i, 0)),
            pl.BlockSpec((bn, H16), lambda i: (i, 0)),
        ],
        out_shape=[
            jax.ShapeDtypeStruct((n, HC), jnp.float32),
            jax.ShapeDtypeStruct((n, H16), jnp.float32),
            jax.ShapeDtypeStruct((n, H16), jnp.float32),
        ],
    )(x, W, asrc, adst)


def _edge_logit_body(ea_ref, we_ref, aedge_ref, e_ref):
    S = _head_sum_matrix(HC, H16, C)
    aw = we_ref[...] * aedge_ref[...]
    V = lax.dot_general(S, aw, (((0,), (0,)), ((), ())),
                        preferred_element_type=jnp.float32)
    e_ref[...] = lax.dot_general(ea_ref[...], V, (((1,), (1,)), ((), ())),
                                 preferred_element_type=jnp.float32)


def _run_edge_logits(ea_p, W_edge, aedge, epad, be):
    grid = epad // be
    return pl.pallas_call(
        _edge_logit_body,
        grid=(grid,),
        in_specs=[
            pl.BlockSpec((be, E_DIM), lambda i: (i, 0)),
            pl.BlockSpec((HC, E_DIM), lambda i: (0, 0)),
            pl.BlockSpec((HC, 1), lambda i: (0, 0)),
        ],
        out_specs=pl.BlockSpec((be, H16), lambda i: (i, 0)),
        out_shape=jax.ShapeDtypeStruct((epad, H16), jnp.float32),
    )(ea_p, W_edge, aedge)


def _sc_pass1_body(src_ref, dst_ref, e_ref, s_ref, d_ref, ones_ref, zeros_ref,
                   p_ref, den_ref, es_ref, cnt_ref,
                   den_sh, es_sh, cnt_sh,
                   sidx, didx, srow, drow, erow, prow, onesv, gsem,
                   nb):
    cid = lax.axis_index("c")
    sid = lax.axis_index("s")
    start = pl.multiple_of(sid * STRIPE, 8)

    pltpu.sync_copy(zeros_ref.at[pl.ds(start, STRIPE)],
                    den_sh.at[pl.ds(start, STRIPE)])
    pltpu.sync_copy(zeros_ref.at[pl.ds(start, STRIPE)],
                    es_sh.at[pl.ds(start, STRIPE)])
    pltpu.sync_copy(zeros_ref.at[pl.ds(start, STRIPE)],
                    cnt_sh.at[pl.ds(start, STRIPE)])
    pltpu.sync_copy(ones_ref, onesv)
    plsc.subcore_barrier()

    base = (cid * NS + sid) * (nb * K)

    def batch_body(b, carry):
        off = pl.multiple_of(base + b * K, 8)
        pltpu.sync_copy(src_ref.at[pl.ds(off, K)], sidx)
        pltpu.sync_copy(dst_ref.at[pl.ds(off, K)], didx)
        pltpu.sync_copy(e_ref.at[pl.ds(off, K)], erow)
        pltpu.async_copy(s_ref.at[sidx], srow, gsem).wait()
        pltpu.async_copy(d_ref.at[didx], drow, gsem).wait()

        def edge_p(j, c2):
            t = srow[j, :] + drow[j, :] + erow[j, :]
            t = jnp.where(t > 0.0, t, 0.2 * t)
            prow[j, :] = jnp.exp(t)
            return c2

        lax.fori_loop(0, K, edge_p, 0)
        pltpu.sync_copy(prow, p_ref.at[pl.ds(off, K)])
        pltpu.sync_copy(prow, den_sh.at[didx], add=True)
        pltpu.sync_copy(erow, es_sh.at[didx], add=True)
        pltpu.sync_copy(onesv, cnt_sh.at[didx], add=True)
        return carry

    lax.fori_loop(0, nb, batch_body, 0)
    plsc.subcore_barrier()

    pltpu.sync_copy(den_sh.at[pl.ds(start, STRIPE)],
                    den_ref.at[cid, pl.ds(start, STRIPE)])
    pltpu.sync_copy(es_sh.at[pl.ds(start, STRIPE)],
                    es_ref.at[cid, pl.ds(start, STRIPE)])
    pltpu.sync_copy(cnt_sh.at[pl.ds(start, STRIPE)],
                    cnt_ref.at[cid, pl.ds(start, STRIPE)])


def _run_sc_pass1(src_p, dst_p, e, s, d_pad, epad):
    nb = epad // (NC * NS * K)
    ones16 = jnp.ones((K, H16), jnp.float32)
    zeros16 = jnp.zeros((NP, H16), jnp.float32)
    mesh = plsc.VectorSubcoreMesh(core_axis_name="c", subcore_axis_name="s")
    body = functools.partial(_sc_pass1_body, nb=nb)
    fn = pl.kernel(
        body,
        out_type=[
            jax.ShapeDtypeStruct((epad, H16), jnp.float32),
            jax.ShapeDtypeStruct((NC, NP, H16), jnp.float32),
            jax.ShapeDtypeStruct((NC, NP, H16), jnp.float32),
            jax.ShapeDtypeStruct((NC, NP, H16), jnp.float32),
        ],
        mesh=mesh,
        compiler_params=pltpu.CompilerParams(use_tc_tiling_on_sc=False),
        scratch_types=[
            pltpu.VMEM_SHARED((NP, H16), jnp.float32),
            pltpu.VMEM_SHARED((NP, H16), jnp.float32),
            pltpu.VMEM_SHARED((NP, H16), jnp.float32),
            pltpu.VMEM((K,), jnp.int32),
            pltpu.VMEM((K,), jnp.int32),
            pltpu.VMEM((K, H16), jnp.float32),
            pltpu.VMEM((K, H16), jnp.float32),
            pltpu.VMEM((K, H16), jnp.float32),
            pltpu.VMEM((K, H16), jnp.float32),
            pltpu.VMEM((K, H16), jnp.float32),
            pltpu.SemaphoreType.DMA,
        ],
    )
    return fn(src_p, dst_p, e, s, d_pad, ones16, zeros16)


def _sc_pass2_body(src_ref, dst_ref, p_ref, xw2_ref, zeros_ref,
                   acc_ref,
                   acc_sh, sidx, didx0, didx1, gidx0, gidx1, prow0, prow1,
                   xrow0, xrow1, gsem0, gsem1,
                   nb):
    cid = lax.axis_index("c")
    sid = lax.axis_index("s")
    start = pl.multiple_of(sid * STRIPE, 8)

    pltpu.sync_copy(zeros_ref.at[pl.ds(start, STRIPE)],
                    acc_sh.at[pl.ds(start, STRIPE)])
    plsc.subcore_barrier()

    base = sid * (nb * K)
    didx = (didx0, didx1)
    gidx = (gidx0, gidx1)
    prow = (prow0, prow1)
    xrow = (xrow0, xrow1)
    gsem = (gsem0, gsem1)

    def start_batch(slot, b):
        off = pl.multiple_of(base + b * K, 8)
        pltpu.sync_copy(src_ref.at[pl.ds(off, K)], sidx)
        pltpu.sync_copy(dst_ref.at[pl.ds(off, K)], didx[slot])
        pltpu.sync_copy(p_ref.at[pl.ds(off, K)], prow[slot])

        def idx_body(i, c2):
            g = sidx[pl.ds(i * 16, 16)] * 2 + cid
            gidx[slot][pl.ds(i * 16, 16)] = g
            return c2

        lax.fori_loop(0, K // 16, idx_body, 0)
        pltpu.async_copy(xw2_ref.at[gidx[slot]], xrow[slot], gsem[slot])

    def finish_batch(slot):
        pltpu.make_async_copy(xw2_ref.at[gidx[slot]], xrow[slot],
                              gsem[slot]).wait()
        xr = xrow[slot]
        pr = prow[slot]

        def edge_body(j, c2):
            pv = pr[j, :]
            for t in range(HALF // 16):
                w = jnp.where(cid == 0, pv[t // 2], pv[H // NC + t // 2])
                xs = xr[j, pl.ds(t * 16, 16)]
                xr[j, pl.ds(t * 16, 16)] = xs * w
            return c2

        lax.fori_loop(0, K, edge_body, 0)
        pltpu.sync_copy(xr, acc_sh.at[didx[slot]], add=True)

    start_batch(0, 0)

    def outer(g, carry):
        start_batch(1, 2 * g + 1)
        finish_batch(0)

        @pl.when(g < (nb // 2) - 1)
        def _():
            start_batch(0, 2 * g + 2)

        finish_batch(1)
        return carry

    lax.fori_loop(0, nb // 2, outer, 0)
    plsc.subcore_barrier()
    pltpu.sync_copy(acc_sh.at[pl.ds(start, STRIPE)],
                    acc_ref.at[cid, pl.ds(start, STRIPE)])


def _run_sc_pass2(src_p, dst_p, p_flat, xw2, epad):
    nb = epad // (NS * K)
    zerosw = jnp.zeros((NP, HALF), jnp.float32)
    mesh = plsc.VectorSubcoreMesh(core_axis_name="c", subcore_axis_name="s")
    body = functools.partial(_sc_pass2_body, nb=nb)
    fn = pl.kernel(
        body,
        out_type=jax.ShapeDtypeStruct((NC, NP, HALF), jnp.float32),
        mesh=mesh,
        compiler_params=pltpu.CompilerParams(use_tc_tiling_on_sc=False),
        scratch_types=[
            pltpu.VMEM_SHARED((NP, HALF), jnp.float32),
            pltpu.VMEM((K,), jnp.int32),
            pltpu.VMEM((K,), jnp.int32),
            pltpu.VMEM((K,), jnp.int32),
            pltpu.VMEM((K,), jnp.int32),
            pltpu.VMEM((K,), jnp.int32),
            pltpu.VMEM((K, H16), jnp.float32),
            pltpu.VMEM((K, H16), jnp.float32),
            pltpu.VMEM((K, HALF), jnp.float32),
            pltpu.VMEM((K, HALF), jnp.float32),
            pltpu.SemaphoreType.DMA,
            pltpu.SemaphoreType.DMA,
        ],
    )
    return fn(src_p, dst_p, p_flat, xw2, zerosw)


def _combine_body(acc0_ref, acc1_ref, den0_ref, den1_ref, es0_ref, es1_ref,
                  cnt0_ref, cnt1_ref, s_ref, d_ref, xw_ref, bias_ref, out_ref):
    den = den0_ref[...] + den1_ref[...]
    es = es0_ref[...] + es1_ref[...]
    deg = cnt0_ref[...] + cnt1_ref[...]
    eloop = es / jnp.maximum(deg, 1.0)
    t = s_ref[...] + d_ref[...] + eloop
    t = jnp.where(t > 0.0, t, 0.2 * t)
    aself = jnp.exp(t)
    scale = 1.0 / ((den + aself + 1e-16) * (deg + 1.0))
    Ex = _head_sum_matrix(HC, H16, C)
    aself_w = lax.dot_general(aself, Ex, (((1,), (1,)), ((), ())),
                              preferred_element_type=jnp.float32)
    scale_w = lax.dot_general(scale, Ex, (((1,), (1,)), ((), ())),
                              preferred_element_type=jnp.float32)
    acc = jnp.concatenate([acc0_ref[...], acc1_ref[...]], axis=1)
    o = (acc + aself_w * xw_ref[...]) * scale_w + bias_ref[...]
    out_ref[...] = jnp.where(o > 0.0, o, jnp.exp(o) - 1.0)


def _run_combine(acc0, acc1, den2, es2, cnt2, s, d, xw, bias_row, n, bn):
    grid = n // bn
    nspec = lambda w: pl.BlockSpec((bn, w), lambda i: (i, 0))
    return pl.pallas_call(
        _combine_body,
        grid=(grid,),
        in_specs=[
            nspec(HALF), nspec(HALF),
            nspec(H16), nspec(H16), nspec(H16), nspec(H16),
            nspec(H16), nspec(H16),
            nspec(H16), nspec(H16),
            nspec(HC),
            pl.BlockSpec((1, HC), lambda i: (0, 0)),
        ],
        out_specs=pl.BlockSpec((bn, HC), lambda i: (i, 0)),
        out_shape=jax.ShapeDtypeStruct((n, HC), jnp.float32),
    )(acc0, acc1, den2[0], den2[1], es2[0], es2[1], cnt2[0], cnt2[1],
      s, d, xw, bias_row)


def kernel(x, edge_index, edge_attr, W, W_edge, att_src, att_dst, att_edge,
           bias):
    n = x.shape[0]
    e_cnt = edge_index.shape[1]
    chunk = NC * NS * K
    epad = ((e_cnt + chunk - 1) // chunk) * chunk
    pad = epad - e_cnt

    src_p = jnp.concatenate(
        [edge_index[0].astype(jnp.int32), jnp.zeros((pad,), jnp.int32)])
    dst_p = jnp.concatenate(
        [edge_index[1].astype(jnp.int32), jnp.full((pad,), n, jnp.int32)])
    ea_p = jnp.concatenate(
        [edge_attr, jnp.zeros((pad, E_DIM), jnp.float32)], axis=0)

    asrc = att_src.reshape(1, HC)
    adst = att_dst.reshape(1, HC)
    aedge = att_edge.reshape(HC, 1)
    bias_row = bias.reshape(1, HC)

    xw, s, d = _run_proj(x, W, asrc, adst, n, bn=2000)
    e = _run_edge_logits(ea_p, W_edge, aedge, epad, be=4096)

    d_pad = jnp.concatenate([d, jnp.zeros((1, H16), jnp.float32)], axis=0)
    p, den2, es2, cnt2 = _run_sc_pass1(src_p, dst_p, e, s, d_pad, epad)

    xw2 = xw.reshape(n * NC, HALF)
    acc2 = _run_sc_pass2(src_p, dst_p, p, xw2, epad)

    return _run_combine(acc2[0, :n], acc2[1, :n], den2[:, :n], es2[:, :n],
                        cnt2[:, :n], s, d, xw, bias_row, n, bn=2000)


# speedup vs baseline: 26.6566x; 1.0926x over previous
import functools

import jax
import jax.numpy as jnp
from jax import lax
from jax.experimental import pallas as pl
from jax.experimental.pallas import tpu as pltpu
from jax.experimental.pallas import tpu_sc as plsc

H = 8
C = 32
HC = H * C
F_IN = 256
E_DIM = 16
H16 = 16
NC = 2
NS = 16
K = 128
HALF = HC // NC
STRIPE = 640
NP = NS * STRIPE


def _head_sum_matrix(n_in, n_heads, group):
    rows = lax.broadcasted_iota(jnp.int32, (n_in, n_heads), 0)
    cols = lax.broadcasted_iota(jnp.int32, (n_in, n_heads), 1)
    return jnp.where(rows // group == cols, 1.0, 0.0).astype(jnp.float32)


def _proj_body(x_ref, w_ref, asrc_ref, adst_ref, xw_ref, s_ref, d_ref):
    xw = lax.dot_general(x_ref[...], w_ref[...], (((1,), (1,)), ((), ())),
                         preferred_element_type=jnp.float32)
    xw_ref[...] = xw
    S = _head_sum_matrix(HC, H16, C)
    s_ref[...] = lax.dot_general(xw * asrc_ref[...], S,
                                 (((1,), (0,)), ((), ())),
                                 preferred_element_type=jnp.float32)
    d_ref[...] = lax.dot_general(xw * adst_ref[...], S,
                                 (((1,), (0,)), ((), ())),
                                 preferred_element_type=jnp.float32)


def _run_proj(x, W, asrc, adst, n, bn):
    grid = n // bn
    return pl.pallas_call(
        _proj_body,
        grid=(grid,),
        in_specs=[
            pl.BlockSpec((bn, F_IN), lambda i: (i, 0)),
            pl.BlockSpec((HC, F_IN), lambda i: (0, 0)),
            pl.BlockSpec((1, HC), lambda i: (0, 0)),
            pl.BlockSpec((1, HC), lambda i: (0, 0)),
        ],
        out_specs=[
            pl.BlockSpec((bn, HC), lambda i: (i, 0)),
            pl.BlockSpec((bn, H16), lambda i: (i, 0)),
            pl.BlockSpec((bn, H16), lambda i: (i, 0)),
        ],
        out_shape=[
            jax.ShapeDtypeStruct((n, HC), jnp.float32),
            jax.ShapeDtypeStruct((n, H16), jnp.float32),
            jax.ShapeDtypeStruct((n, H16), jnp.float32),
        ],
    )(x, W, asrc, adst)


def _edge_logit_body(ea_ref, we_ref, aedge_ref, e_ref):
    S = _head_sum_matrix(HC, H16, C)
    aw = we_ref[...] * aedge_ref[...]
    V = lax.dot_general(S, aw, (((0,), (0,)), ((), ())),
                        preferred_element_type=jnp.float32)
    e_ref[...] = lax.dot_general(ea_ref[...], V, (((1,), (1,)), ((), ())),
                                 preferred_element_type=jnp.float32)


def _run_edge_logits(ea_p, W_edge, aedge, epad, be):
    grid = epad // be
    return pl.pallas_call(
        _edge_logit_body,
        grid=(grid,),
        in_specs=[
            pl.BlockSpec((be, E_DIM), lambda i: (i, 0)),
            pl.BlockSpec((HC, E_DIM), lambda i: (0, 0)),
            pl.BlockSpec((HC, 1), lambda i: (0, 0)),
        ],
        out_specs=pl.BlockSpec((be, H16), lambda i: (i, 0)),
        out_shape=jax.ShapeDtypeStruct((epad, H16), jnp.float32),
    )(ea_p, W_edge, aedge)


def _sc_pass1_body(src_ref, dst_ref, e_ref, s_ref, d_ref, ones_ref, zeros_ref,
                   p_ref, den_ref, es_ref, cnt_ref,
                   den_sh, es_sh, cnt_sh,
                   sidx0, sidx1, didx0, didx1, srow0, srow1, drow0, drow1,
                   erow0, erow1, prow0, prow1, onesv,
                   gsem0, gsem1, ssem0, ssem1,
                   nb):
    cid = lax.axis_index("c")
    sid = lax.axis_index("s")
    start = pl.multiple_of(sid * STRIPE, 8)

    pltpu.sync_copy(zeros_ref.at[pl.ds(start, STRIPE)],
                    den_sh.at[pl.ds(start, STRIPE)])
    pltpu.sync_copy(zeros_ref.at[pl.ds(start, STRIPE)],
                    es_sh.at[pl.ds(start, STRIPE)])
    pltpu.sync_copy(zeros_ref.at[pl.ds(start, STRIPE)],
                    cnt_sh.at[pl.ds(start, STRIPE)])
    pltpu.sync_copy(ones_ref, onesv)
    plsc.subcore_barrier()

    base = (cid * NS + sid) * (nb * K)
    sidx = (sidx0, sidx1)
    didx = (didx0, didx1)
    srow = (srow0, srow1)
    drow = (drow0, drow1)
    erow = (erow0, erow1)
    prow = (prow0, prow1)
    gsem = (gsem0, gsem1)
    ssem = (ssem0, ssem1)

    def start_batch(slot, b):
        off = pl.multiple_of(base + b * K, 8)
        pltpu.sync_copy(src_ref.at[pl.ds(off, K)], sidx[slot])
        pltpu.sync_copy(dst_ref.at[pl.ds(off, K)], didx[slot])
        pltpu.sync_copy(e_ref.at[pl.ds(off, K)], erow[slot])
        pltpu.async_copy(s_ref.at[sidx[slot]], srow[slot], gsem[slot])
        pltpu.async_copy(d_ref.at[didx[slot]], drow[slot], gsem[slot])

    def finish_batch(slot, b):
        off = pl.multiple_of(base + b * K, 8)
        pltpu.make_async_copy(s_ref.at[sidx[slot]], srow[slot],
                              gsem[slot]).wait()
        pltpu.make_async_copy(d_ref.at[didx[slot]], drow[slot],
                              gsem[slot]).wait()

        def edge_p(j, c2):
            t = srow[slot][j, :] + drow[slot][j, :] + erow[slot][j, :]
            t = jnp.where(t > 0.0, t, 0.2 * t)
            prow[slot][j, :] = jnp.exp(t)
            return c2

        lax.fori_loop(0, K, edge_p, 0)
        pltpu.sync_copy(prow[slot], p_ref.at[pl.ds(off, K)])
        pltpu.sync_copy(prow[slot], den_sh.at[didx[slot]], add=True)
        pltpu.sync_copy(erow[slot], es_sh.at[didx[slot]], add=True)
        pltpu.sync_copy(onesv, cnt_sh.at[didx[slot]], add=True)

    start_batch(0, 0)

    def outer(g, carry):
        start_batch(1, 2 * g + 1)
        finish_batch(0, 2 * g)

        @pl.when(g < (nb // 2) - 1)
        def _():
            start_batch(0, 2 * g + 2)

        finish_batch(1, 2 * g + 1)
        return carry

    lax.fori_loop(0, nb // 2, outer, 0)
    plsc.subcore_barrier()

    pltpu.sync_copy(den_sh.at[pl.ds(start, STRIPE)],
                    den_ref.at[cid, pl.ds(start, STRIPE)])
    pltpu.sync_copy(es_sh.at[pl.ds(start, STRIPE)],
                    es_ref.at[cid, pl.ds(start, STRIPE)])
    pltpu.sync_copy(cnt_sh.at[pl.ds(start, STRIPE)],
                    cnt_ref.at[cid, pl.ds(start, STRIPE)])


def _run_sc_pass1(src_p, dst_p, e, s, d_pad, epad):
    nb = epad // (NC * NS * K)
    ones16 = jnp.ones((K, H16), jnp.float32)
    zeros16 = jnp.zeros((NP, H16), jnp.float32)
    mesh = plsc.VectorSubcoreMesh(core_axis_name="c", subcore_axis_name="s")
    body = functools.partial(_sc_pass1_body, nb=nb)
    fn = pl.kernel(
        body,
        out_type=[
            jax.ShapeDtypeStruct((epad, H16), jnp.float32),
            jax.ShapeDtypeStruct((NC, NP, H16), jnp.float32),
            jax.ShapeDtypeStruct((NC, NP, H16), jnp.float32),
            jax.ShapeDtypeStruct((NC, NP, H16), jnp.float32),
        ],
        mesh=mesh,
        compiler_params=pltpu.CompilerParams(use_tc_tiling_on_sc=False),
        scratch_types=[
            pltpu.VMEM_SHARED((NP, H16), jnp.float32),
            pltpu.VMEM_SHARED((NP, H16), jnp.float32),
            pltpu.VMEM_SHARED((NP, H16), jnp.float32),
            pltpu.VMEM((K,), jnp.int32),
            pltpu.VMEM((K,), jnp.int32),
            pltpu.VMEM((K,), jnp.int32),
            pltpu.VMEM((K,), jnp.int32),
            pltpu.VMEM((K, H16), jnp.float32),
            pltpu.VMEM((K, H16), jnp.float32),
            pltpu.VMEM((K, H16), jnp.float32),
            pltpu.VMEM((K, H16), jnp.float32),
            pltpu.VMEM((K, H16), jnp.float32),
            pltpu.VMEM((K, H16), jnp.float32),
            pltpu.VMEM((K, H16), jnp.float32),
            pltpu.VMEM((K, H16), jnp.float32),
            pltpu.VMEM((K, H16), jnp.float32),
            pltpu.SemaphoreType.DMA,
            pltpu.SemaphoreType.DMA,
            pltpu.SemaphoreType.DMA,
            pltpu.SemaphoreType.DMA,
        ],
    )
    return fn(src_p, dst_p, e, s, d_pad, ones16, zeros16)


def _sc_pass2_body(src_ref, dst_ref, p_ref, xw2_ref, zeros_ref,
                   acc_ref,
                   acc_sh, sidx, didx0, didx1, gidx0, gidx1, prow0, prow1,
                   xrow0, xrow1, gsem0, gsem1, ssem0, ssem1,
                   nb):
    cid = lax.axis_index("c")
    sid = lax.axis_index("s")
    start = pl.multiple_of(sid * STRIPE, 8)

    pltpu.sync_copy(zeros_ref.at[pl.ds(start, STRIPE)],
                    acc_sh.at[pl.ds(start, STRIPE)])
    plsc.subcore_barrier()

    base = sid * (nb * K)
    didx = (didx0, didx1)
    gidx = (gidx0, gidx1)
    prow = (prow0, prow1)
    xrow = (xrow0, xrow1)
    gsem = (gsem0, gsem1)
    ssem = (ssem0, ssem1)

    def start_batch(slot, b):
        off = pl.multiple_of(base + b * K, 8)
        pltpu.sync_copy(src_ref.at[pl.ds(off, K)], sidx)
        pltpu.sync_copy(dst_ref.at[pl.ds(off, K)], didx[slot])
        pltpu.sync_copy(p_ref.at[pl.ds(off, K)], prow[slot])

        def idx_body(i, c2):
            g = sidx[pl.ds(i * 16, 16)] * 2 + cid
            gidx[slot][pl.ds(i * 16, 16)] = g
            return c2

        lax.fori_loop(0, K // 16, idx_body, 0)
        pltpu.async_copy(xw2_ref.at[gidx[slot]], xrow[slot], gsem[slot])

    def finish_batch(slot):
        pltpu.make_async_copy(xw2_ref.at[gidx[slot]], xrow[slot],
                              gsem[slot]).wait()
        xr = xrow[slot]
        pr = prow[slot]

        def edge_body(j, c2):
            pv = pr[j, :]
            for t in range(HALF // 16):
                w = jnp.where(cid == 0, pv[t // 2], pv[H // NC + t // 2])
                xs = xr[j, pl.ds(t * 16, 16)]
                xr[j, pl.ds(t * 16, 16)] = xs * w
            return c2

        lax.fori_loop(0, K, edge_body, 0)
        pltpu.sync_copy(xr, acc_sh.at[didx[slot]], add=True)

    start_batch(0, 0)

    def outer(g, carry):
        start_batch(1, 2 * g + 1)
        finish_batch(0)

        @pl.when(g < (nb // 2) - 1)
        def _():
            start_batch(0, 2 * g + 2)

        finish_batch(1)
        return carry

    lax.fori_loop(0, nb // 2, outer, 0)
    plsc.subcore_barrier()
    pltpu.sync_copy(acc_sh.at[pl.ds(start, STRIPE)],
                    acc_ref.at[cid, pl.ds(start, STRIPE)])


def _run_sc_pass2(src_p, dst_p, p_flat, xw2, epad):
    nb = epad // (NS * K)
    zerosw = jnp.zeros((NP, HALF), jnp.float32)
    mesh = plsc.VectorSubcoreMesh(core_axis_name="c", subcore_axis_name="s")
    body = functools.partial(_sc_pass2_body, nb=nb)
    fn = pl.kernel(
        body,
        out_type=jax.ShapeDtypeStruct((NC, NP, HALF), jnp.float32),
        mesh=mesh,
        compiler_params=pltpu.CompilerParams(use_tc_tiling_on_sc=False),
        scratch_types=[
            pltpu.VMEM_SHARED((NP, HALF), jnp.float32),
            pltpu.VMEM((K,), jnp.int32),
            pltpu.VMEM((K,), jnp.int32),
            pltpu.VMEM((K,), jnp.int32),
            pltpu.VMEM((K,), jnp.int32),
            pltpu.VMEM((K,), jnp.int32),
            pltpu.VMEM((K, H16), jnp.float32),
            pltpu.VMEM((K, H16), jnp.float32),
            pltpu.VMEM((K, HALF), jnp.float32),
            pltpu.VMEM((K, HALF), jnp.float32),
            pltpu.SemaphoreType.DMA,
            pltpu.SemaphoreType.DMA,
            pltpu.SemaphoreType.DMA,
            pltpu.SemaphoreType.DMA,
        ],
    )
    return fn(src_p, dst_p, p_flat, xw2, zerosw)


def _combine_body(acc0_ref, acc1_ref, den0_ref, den1_ref, es0_ref, es1_ref,
                  cnt0_ref, cnt1_ref, s_ref, d_ref, xw_ref, bias_ref, out_ref):
    den = den0_ref[...] + den1_ref[...]
    es = es0_ref[...] + es1_ref[...]
    deg = cnt0_ref[...] + cnt1_ref[...]
    eloop = es / jnp.maximum(deg, 1.0)
    t = s_ref[...] + d_ref[...] + eloop
    t = jnp.where(t > 0.0, t, 0.2 * t)
    aself = jnp.exp(t)
    scale = 1.0 / ((den + aself + 1e-16) * (deg + 1.0))
    Ex = _head_sum_matrix(HC, H16, C)
    aself_w = lax.dot_general(aself, Ex, (((1,), (1,)), ((), ())),
                              preferred_element_type=jnp.float32)
    scale_w = lax.dot_general(scale, Ex, (((1,), (1,)), ((), ())),
                              preferred_element_type=jnp.float32)
    acc = jnp.concatenate([acc0_ref[...], acc1_ref[...]], axis=1)
    o = (acc + aself_w * xw_ref[...]) * scale_w + bias_ref[...]
    out_ref[...] = jnp.where(o > 0.0, o, jnp.exp(o) - 1.0)


def _run_combine(acc0, acc1, den2, es2, cnt2, s, d, xw, bias_row, n, bn):
    grid = n // bn
    nspec = lambda w: pl.BlockSpec((bn, w), lambda i: (i, 0))
    return pl.pallas_call(
        _combine_body,
        grid=(grid,),
        in_specs=[
            nspec(HALF), nspec(HALF),
            nspec(H16), nspec(H16), nspec(H16), nspec(H16),
            nspec(H16), nspec(H16),
            nspec(H16), nspec(H16),
            nspec(HC),
            pl.BlockSpec((1, HC), lambda i: (0, 0)),
        ],
        out_specs=pl.BlockSpec((bn, HC), lambda i: (i, 0)),
        out_shape=jax.ShapeDtypeStruct((n, HC), jnp.float32),
    )(acc0, acc1, den2[0], den2[1], es2[0], es2[1], cnt2[0], cnt2[1],
      s, d, xw, bias_row)


def kernel(x, edge_index, edge_attr, W, W_edge, att_src, att_dst, att_edge,
           bias):
    n = x.shape[0]
    e_cnt = edge_index.shape[1]
    chunk = NC * NS * K
    epad = ((e_cnt + chunk - 1) // chunk) * chunk
    pad = epad - e_cnt

    src_p = jnp.concatenate(
        [edge_index[0].astype(jnp.int32), jnp.zeros((pad,), jnp.int32)])
    dst_p = jnp.concatenate(
        [edge_index[1].astype(jnp.int32), jnp.full((pad,), n, jnp.int32)])
    ea_p = jnp.concatenate(
        [edge_attr, jnp.zeros((pad, E_DIM), jnp.float32)], axis=0)

    asrc = att_src.reshape(1, HC)
    adst = att_dst.reshape(1, HC)
    aedge = att_edge.reshape(HC, 1)
    bias_row = bias.reshape(1, HC)

    xw, s, d = _run_proj(x, W, asrc, adst, n, bn=2000)
    e = _run_edge_logits(ea_p, W_edge, aedge, epad, be=4096)

    d_pad = jnp.concatenate([d, jnp.zeros((1, H16), jnp.float32)], axis=0)
    p, den2, es2, cnt2 = _run_sc_pass1(src_p, dst_p, e, s, d_pad, epad)

    xw2 = xw.reshape(n * NC, HALF)
    acc2 = _run_sc_pass2(src_p, dst_p, p, xw2, epad)

    return _run_combine(acc2[0, :n], acc2[1, :n], den2[:, :n], es2[:, :n],
                        cnt2[:, :n], s, d, xw, bias_row, n, bn=2000)
